# baseline (speedup 1.0000x reference)
import functools

import jax
import jax.numpy as jnp
from jax import lax
from jax.experimental import pallas as pl
from jax.experimental.pallas import tpu as pltpu
from jax.experimental.pallas import tpu_sc as plsc

_NC, _NS = 2, 16
_NW = _NC * _NS


def _make_sc_gather(B, F, Fd):
    b_per_w = B // _NW
    mesh = plsc.VectorSubcoreMesh(
        core_axis_name="c", subcore_axis_name="s", num_cores=_NC)

    @functools.partial(
        pl.kernel,
        out_type=(
            jax.ShapeDtypeStruct((B, F), jnp.float32),
            jax.ShapeDtypeStruct((B, F), jnp.float32),
            jax.ShapeDtypeStruct((B, Fd), jnp.float32),
            jax.ShapeDtypeStruct((B,), jnp.float32),
        ),
        mesh=mesh,
        scratch_types=[
            pltpu.VMEM((b_per_w,), jnp.int32),
            pltpu.VMEM((b_per_w,), jnp.int32),
            pltpu.VMEM((b_per_w, F), jnp.float32),
            pltpu.VMEM((b_per_w, F), jnp.float32),
            pltpu.VMEM((b_per_w, Fd), jnp.float32),
            pltpu.VMEM((b_per_w,), jnp.float32),
            pltpu.SemaphoreType.DMA,
        ],
        compiler_params=pltpu.CompilerParams(use_tc_tiling_on_sc=False),
    )
    def gather_kernel(user_hbm, item_hbm, bi_hbm, gu_hbm, gi_hbm, tu_hbm,
                      gu_out, gi_out, tu_out, bi_out,
                      uidx, iidx, gu_v, gi_v, tu_v, bi_v, sem):
        wid = lax.axis_index("s") * _NC + lax.axis_index("c")
        base = wid * b_per_w
        pltpu.sync_copy(user_hbm.at[pl.ds(base, b_per_w)], uidx)
        pltpu.sync_copy(item_hbm.at[pl.ds(base, b_per_w)], iidx)
        c1 = pltpu.async_copy(gu_hbm.at[uidx], gu_v, sem)
        c2 = pltpu.async_copy(gi_hbm.at[iidx], gi_v, sem)
        c3 = pltpu.async_copy(tu_hbm.at[uidx], tu_v, sem)
        c4 = pltpu.async_copy(bi_hbm.at[iidx], bi_v, sem)
        c1.wait()
        c2.wait()
        c3.wait()
        c4.wait()
        pltpu.sync_copy(gu_v, gu_out.at[pl.ds(base, b_per_w)])
        pltpu.sync_copy(gi_v, gi_out.at[pl.ds(base, b_per_w)])
        pltpu.sync_copy(tu_v, tu_out.at[pl.ds(base, b_per_w)])
        pltpu.sync_copy(bi_v, bi_out.at[pl.ds(base, b_per_w)])

    return gather_kernel


def _make_tc_combine(B, K, F, N, blk):
    def body(feat_ref, ew_ref, gu_ref, gi_ref, thp_ref, beta_ref, out_ref):
        r = jnp.dot(feat_ref[...], ew_ref[...],
                    preferred_element_type=jnp.float32)
        out_ref[...] = (beta_ref[...]
                        + jnp.sum(gu_ref[...] * gi_ref[...], axis=1)
                        + jnp.sum(thp_ref[...] * r, axis=1))

    return pl.pallas_call(
        body,
        grid=(B // blk,),
        in_specs=[
            pl.BlockSpec((blk, K), lambda b: (b, 0)),
            pl.BlockSpec((K, N), lambda b: (0, 0)),
            pl.BlockSpec((blk, F), lambda b: (b, 0)),
            pl.BlockSpec((blk, F), lambda b: (b, 0)),
            pl.BlockSpec((blk, N), lambda b: (b, 0)),
            pl.BlockSpec((blk,), lambda b: (b,)),
        ],
        out_specs=pl.BlockSpec((blk,), lambda b: (b,)),
        out_shape=jax.ShapeDtypeStruct((B,), jnp.float32),
        compiler_params=pltpu.CompilerParams(
            dimension_semantics=("arbitrary",)),
    )


def kernel(user, item, feature_i, Bi, Gu, Gi, Bp, Tu, E):
    B = user.shape[0]
    K, Fd = E.shape
    F = Gu.shape[1]
    N = 64

    gamma_u, gamma_i, theta_u, beta_i = _make_sc_gather(B, F, Fd)(
        user, item, Bi, Gu, Gi, Tu)

    Ew = jnp.concatenate(
        [E, Bp, jnp.zeros((K, N - Fd - 1), jnp.float32)], axis=1)
    thp = jnp.concatenate(
        [theta_u, jnp.ones((B, 1), jnp.float32),
         jnp.zeros((B, N - Fd - 1), jnp.float32)], axis=1)

    xui = _make_tc_combine(B, K, F, N, 512)(
        feature_i, Ew, gamma_u, gamma_i, thp, beta_i)

    return (xui, gamma_u, gamma_i, feature_i, theta_u, beta_i)
